# baseline (speedup 1.0000x reference)
import functools

import jax
import jax.numpy as jnp
from jax import lax
from jax.experimental import pallas as pl
from jax.experimental.pallas import tpu as pltpu
from jax.experimental.pallas import tpu_sc as plsc

_VOCAB = 1000000
_D = 64
_B = 16384
_K = 10

_NC = 2
_NS = 16
_NW = _NC * _NS
_BPW = _B // _NW
_CH = 128
_NCHUNK = _BPW // _CH
_NGRP = _CH // 16


def _sc_scores(in_idx, ctx_idx, noise_idx, w_in, w_ctx):
    mesh = plsc.VectorSubcoreMesh(core_axis_name="c", subcore_axis_name="s")

    @functools.partial(
        pl.kernel,
        out_type=jax.ShapeDtypeStruct((_NW, 1 + _K, _BPW), jnp.float32),
        mesh=mesh,
        compiler_params=pltpu.CompilerParams(
            needs_layout_passes=False, use_tc_tiling_on_sc=False),
        scratch_types=[
            pltpu.VMEM((_NCHUNK, _CH), jnp.int32),
            pltpu.VMEM((_NCHUNK, _CH), jnp.int32),
            pltpu.VMEM((_NCHUNK * _K, _CH), jnp.int32),
            pltpu.VMEM((_CH, _D), jnp.float32),
            pltpu.VMEM((_CH, _D), jnp.float32),
            pltpu.VMEM((_CH * _K, _D), jnp.float32),
            pltpu.VMEM((1 + _K, _BPW), jnp.float32),
            pltpu.SemaphoreType.DMA,
        ],
    )
    def kern(in_idx_hbm, ctx_idx_hbm, noise_idx_hbm, win_hbm, wctx_hbm,
             out_hbm, iin_v, ictx_v, inoi_v, cen_v, ctx_v, noi_v, sc_v, sem):
        wid = lax.axis_index("s") * _NC + lax.axis_index("c")
        pltpu.sync_copy(in_idx_hbm.at[pl.ds(wid * _NCHUNK, _NCHUNK)], iin_v)
        pltpu.sync_copy(ctx_idx_hbm.at[pl.ds(wid * _NCHUNK, _NCHUNK)], ictx_v)
        pltpu.sync_copy(
            noise_idx_hbm.at[pl.ds(wid * _NCHUNK * _K, _NCHUNK * _K)], inoi_v)

        lanes = lax.iota(jnp.int32, 16)

        for c in range(_NCHUNK):
            cps = [
                pltpu.async_copy(win_hbm.at[iin_v.at[c]], cen_v, sem),
                pltpu.async_copy(wctx_hbm.at[ictx_v.at[c]], ctx_v, sem),
            ]
            for j in range(_K):
                cps.append(pltpu.async_copy(
                    wctx_hbm.at[inoi_v.at[c * _K + j]],
                    noi_v.at[pl.ds(j * _CH, _CH)], sem))
            for cp in cps:
                cp.wait()

            for g in range(_NGRP):
                row = lanes + g * 16
                nrow0 = lanes * _K + g * 16 * _K

                def dbody(d, accs, row=row, nrow0=nrow0):
                    col = jnp.full((16,), d, jnp.int32)
                    cen_d = plsc.load_gather(cen_v, [row, col])
                    ctx_d = plsc.load_gather(ctx_v, [row, col])
                    out = [accs[0] + cen_d * ctx_d]
                    for k in range(_K):
                        n_d = plsc.load_gather(noi_v, [nrow0 + k, col])
                        out.append(accs[1 + k] + n_d * cen_d)
                    return tuple(out)

                zero = jnp.zeros((16,), jnp.float32)
                accs = lax.fori_loop(0, _D, dbody, (zero,) * (1 + _K))

                base = c * _CH + g * 16
                sc_v[0, pl.ds(base, 16)] = accs[0]
                for k in range(_K):
                    sc_v[1 + k, pl.ds(base, 16)] = -accs[1 + k]

        pltpu.sync_copy(sc_v, out_hbm.at[wid])

    return kern(in_idx, ctx_idx, noise_idx, w_in, w_ctx)


def _tc_loss(scores):
    def body(s_ref, o_ref):
        s = s_ref[...]
        ls = jnp.minimum(s, 0.0) - jnp.log1p(jnp.exp(-jnp.abs(s)))
        o_ref[0, 0] = jnp.sum(ls) * (-1.0 / _B)

    out = pl.pallas_call(
        body,
        out_shape=jax.ShapeDtypeStruct((1, 1), jnp.float32),
        out_specs=pl.BlockSpec(memory_space=pltpu.SMEM),
    )(scores)
    return out[0, 0]


def kernel(input_word, context_word, noise_words, W_in, W_ctx):
    in_idx = input_word.astype(jnp.int32).reshape(_NW * _NCHUNK, _CH)
    ctx_idx = context_word.astype(jnp.int32).reshape(_NW * _NCHUNK, _CH)
    noise_idx = noise_words.astype(jnp.int32).reshape(_NW * _NCHUNK * _K, _CH)
    scores = _sc_scores(in_idx, ctx_idx, noise_idx, W_in, W_ctx)
    return _tc_loss(scores.reshape(_NW * (1 + _K), _BPW))


# speedup vs baseline: 1.1435x; 1.1435x over previous
import functools

import jax
import jax.numpy as jnp
from jax import lax
from jax.experimental import pallas as pl
from jax.experimental.pallas import tpu as pltpu
from jax.experimental.pallas import tpu_sc as plsc

_VOCAB = 1000000
_D = 64
_B = 16384
_K = 10
_P = 1 + _K
_L = 16

_NC = 2
_NS = 16
_NW = _NC * _NS
_BPW = _B // _NW
_CH = 64
_NCHUNK = _BPW // _CH
_OUTW = _CH * _P * _L


def _sc_partials(in_idx, ctx_idx, noise_idx, w_in, w_ctx):
    mesh = plsc.VectorSubcoreMesh(core_axis_name="c", subcore_axis_name="s")

    @functools.partial(
        pl.kernel,
        out_type=jax.ShapeDtypeStruct((_NW, _NCHUNK, _OUTW), jnp.float32),
        mesh=mesh,
        compiler_params=pltpu.CompilerParams(
            needs_layout_passes=False, use_tc_tiling_on_sc=False),
        scratch_types=[
            pltpu.VMEM((_NCHUNK, _CH), jnp.int32),
            pltpu.VMEM((_NCHUNK, _CH), jnp.int32),
            pltpu.VMEM((_NCHUNK * _K, _CH), jnp.int32),
            pltpu.VMEM((2, _CH, _D), jnp.float32),
            pltpu.VMEM((2, _CH, _D), jnp.float32),
            pltpu.VMEM((2, _CH * _K, _D), jnp.float32),
            pltpu.VMEM((2, _OUTW), jnp.float32),
            pltpu.SemaphoreType.DMA,
            pltpu.SemaphoreType.DMA,
            pltpu.SemaphoreType.DMA,
            pltpu.SemaphoreType.DMA,
        ],
    )
    def kern(in_idx_hbm, ctx_idx_hbm, noise_idx_hbm, win_hbm, wctx_hbm,
             out_hbm, iin_v, ictx_v, inoi_v, cen_v, ctx_v, noi_v, out_v,
             gsem0, gsem1, osem0, osem1):
        wid = lax.axis_index("s") * _NC + lax.axis_index("c")
        pltpu.sync_copy(in_idx_hbm.at[pl.ds(wid * _NCHUNK, _NCHUNK)], iin_v)
        pltpu.sync_copy(ctx_idx_hbm.at[pl.ds(wid * _NCHUNK, _NCHUNK)], ictx_v)
        pltpu.sync_copy(
            noise_idx_hbm.at[pl.ds(wid * _NCHUNK * _K, _NCHUNK * _K)], inoi_v)

        gsems = (gsem0, gsem1)
        osems = (osem0, osem1)

        def fire_gathers(c):
            b = c % 2
            sem = gsems[b]
            cps = [
                pltpu.async_copy(win_hbm.at[iin_v.at[c]], cen_v.at[b], sem),
                pltpu.async_copy(wctx_hbm.at[ictx_v.at[c]], ctx_v.at[b], sem),
            ]
            for j in range(_K):
                cps.append(pltpu.async_copy(
                    wctx_hbm.at[inoi_v.at[c * _K + j]],
                    noi_v.at[b, pl.ds(j * _CH, _CH)], sem))
            return cps

        pending = {0: fire_gathers(0)}
        out_cps = {}

        for c in range(_NCHUNK):
            b = c % 2
            for cp in pending.pop(c):
                cp.wait()
            if c + 1 < _NCHUNK:
                pending[c + 1] = fire_gathers(c + 1)
            if c >= 2:
                out_cps.pop(c - 2).wait()

            cenb, ctxb, noib, outb = cen_v.at[b], ctx_v.at[b], noi_v.at[b], out_v.at[b]

            def rbody(r, carry, cenb=cenb, ctxb=ctxb, noib=noib, outb=outb):
                cen = [cenb[r, pl.ds(t * _L, _L)] for t in range(_D // _L)]
                ncen = [-v for v in cen]
                p = cen[0] * ctxb[r, pl.ds(0, _L)]
                for t in range(1, _D // _L):
                    p += cen[t] * ctxb[r, pl.ds(t * _L, _L)]
                outb[pl.ds(r * (_P * _L), _L)] = p
                for k in range(_K):
                    nrow = r * _K + k
                    q = ncen[0] * noib[nrow, pl.ds(0, _L)]
                    for t in range(1, _D // _L):
                        q += ncen[t] * noib[nrow, pl.ds(t * _L, _L)]
                    outb[pl.ds(r * (_P * _L) + (1 + k) * _L, _L)] = q
                return carry

            lax.fori_loop(0, _CH, rbody, 0, unroll=2)

            out_cps[c] = pltpu.async_copy(outb, out_hbm.at[wid, c], osems[b])

        out_cps.pop(_NCHUNK - 2).wait()
        out_cps.pop(_NCHUNK - 1).wait()

    return kern(in_idx, ctx_idx, noise_idx, w_in, w_ctx)


def _tc_loss(partials):
    rows = _B * _P * _L // 128
    nsteps = 16
    blk = rows // nsteps

    def body(x_ref, o_ref):
        i = pl.program_id(0)
        x = x_ref[...]
        g = lax.broadcasted_iota(jnp.int32, (128, 8), 0) // _L
        j = lax.broadcasted_iota(jnp.int32, (128, 8), 1)
        m = jnp.where(g == j, 1.0, 0.0)
        scores = jax.lax.dot(x, m, precision=jax.lax.Precision.HIGHEST)
        ls = jnp.minimum(scores, 0.0) - jnp.log1p(jnp.exp(-jnp.abs(scores)))
        part = jnp.sum(ls) * (-1.0 / _B)

        @pl.when(i == 0)
        def _():
            o_ref[0, 0] = part

        @pl.when(i > 0)
        def _():
            o_ref[0, 0] += part

    out = pl.pallas_call(
        body,
        grid=(nsteps,),
        in_specs=[pl.BlockSpec((blk, 128), lambda i: (i, 0))],
        out_shape=jax.ShapeDtypeStruct((1, 1), jnp.float32),
        out_specs=pl.BlockSpec(
            (1, 1), lambda i: (0, 0), memory_space=pltpu.SMEM),
    )(partials)
    return out[0, 0]


def kernel(input_word, context_word, noise_words, W_in, W_ctx):
    in_idx = input_word.astype(jnp.int32).reshape(_NW * _NCHUNK, _CH)
    ctx_idx = context_word.astype(jnp.int32).reshape(_NW * _NCHUNK, _CH)
    noise_idx = noise_words.astype(jnp.int32).reshape(_NW * _NCHUNK * _K, _CH)
    parts = _sc_partials(in_idx, ctx_idx, noise_idx, W_in, W_ctx)
    return _tc_loss(parts.reshape(_B * _P * _L // 128, 128))
